# baseline (speedup 1.0000x reference)
import functools

import jax
import jax.numpy as jnp
from jax.experimental import pallas as pl
from jax.experimental.pallas import tpu as pltpu

_EPS = 1e-5


def _conv9_body(x_ref, w_ref, a_ref, b_ref,
                o_ref, sm_ref, sq_ref,
                xp_ref, col_ref,
                *, D, H, W, XM, BN, apply_affine, planar_in):
    C = x_ref.shape[1]
    DHW = D * H * W
    HW = H * W
    SEG = DHW + 2 * XM
    CSEG = DHW + 2 * HW

    zx = jnp.zeros((C, XM), jnp.bfloat16)
    zc = jnp.zeros((9 * C, HW), jnp.bfloat16)
    for b in range(BN):
        xp_ref[:, b * SEG:b * SEG + XM] = zx
        xp_ref[:, b * SEG + XM + DHW:(b + 1) * SEG] = zx
        col_ref[:, b * CSEG:b * CSEG + HW] = zc
        col_ref[:, b * CSEG + HW + DHW:(b + 1) * CSEG] = zc

    for b in range(BN):
        base = b * SEG + XM
        if planar_in:
            for d in range(D):
                xp_ref[:, base + d * HW:base + (d + 1) * HW] = \
                    x_ref[b, :, d, :].astype(jnp.bfloat16)
        else:
            x = x_ref[b]
            if apply_affine:
                x = jnp.maximum(
                    x.astype(jnp.float32) * a_ref[...] + b_ref[...], 0.0)
            xp_ref[:, base:base + DHW] = x.astype(jnp.bfloat16)

    pos = jax.lax.broadcasted_iota(jnp.int32, (1, DHW), 1)
    h_pos = (pos // W) % H
    w_pos = pos % W
    hm = {-1: (h_pos >= 1).astype(jnp.bfloat16),
          0: None,
          1: (h_pos <= H - 2).astype(jnp.bfloat16)}
    wm = {-1: (w_pos >= 1).astype(jnp.bfloat16),
          0: None,
          1: (w_pos <= W - 2).astype(jnp.bfloat16)}

    for b in range(BN):
        base = b * SEG + XM
        t = 0
        for oh in (-1, 0, 1):
            for ow in (-1, 0, 1):
                si = oh * W + ow
                xs = xp_ref[:, base + si:base + si + DHW]
                m = hm[oh] if wm[ow] is None else (
                    wm[ow] if hm[oh] is None else hm[oh] * wm[ow])
                if m is not None:
                    xs = xs * m
                col_ref[t * C:(t + 1) * C,
                        b * CSEG + HW:b * CSEG + HW + DHW] = xs
                t += 1

    Cout = o_ref.shape[1]
    p = jnp.dot(w_ref[...], col_ref[...],
                preferred_element_type=jnp.float32)
    for b in range(BN):
        cb = b * CSEG
        acc = (p[0:Cout, cb:cb + DHW]
               + p[Cout:2 * Cout, cb + HW:cb + HW + DHW]
               + p[2 * Cout:3 * Cout, cb + 2 * HW:cb + 2 * HW + DHW])
        o_ref[b] = acc.astype(jnp.bfloat16)
        sm_ref[b] = jnp.sum(acc, axis=1, keepdims=True)
        sq_ref[b] = jnp.sum(acc * acc, axis=1, keepdims=True)


def _conv9(x_in, w3, D, H, W, scale, shift, apply_affine, planar_in=False,
           BN=2):
    N, C = x_in.shape[:2]
    DHW = D * H * W
    Cout = w3.shape[0] // 3
    HW = H * W
    XM = 128
    assert XM >= W + 1 and N % BN == 0

    if planar_in:
        x_spec = pl.BlockSpec((BN, C, D, HW), lambda n: (n, 0, 0, 0))
    else:
        x_spec = pl.BlockSpec((BN, C, DHW), lambda n: (n, 0, 0))
    body = functools.partial(_conv9_body, D=D, H=H, W=W, XM=XM, BN=BN,
                             apply_affine=apply_affine, planar_in=planar_in)
    return pl.pallas_call(
        body,
        out_shape=(
            jax.ShapeDtypeStruct((N, Cout, DHW), jnp.bfloat16),
            jax.ShapeDtypeStruct((N, Cout, 1), jnp.float32),
            jax.ShapeDtypeStruct((N, Cout, 1), jnp.float32),
        ),
        grid=(N // BN,),
        in_specs=[
            x_spec,
            pl.BlockSpec((3 * Cout, 9 * C), lambda n: (0, 0)),
            pl.BlockSpec((C, 1), lambda n: (0, 0)),
            pl.BlockSpec((C, 1), lambda n: (0, 0)),
        ],
        out_specs=(
            pl.BlockSpec((BN, Cout, DHW), lambda n: (n, 0, 0)),
            pl.BlockSpec((BN, Cout, 1), lambda n: (n, 0, 0)),
            pl.BlockSpec((BN, Cout, 1), lambda n: (n, 0, 0)),
        ),
        scratch_shapes=[
            pltpu.VMEM((C, BN * (DHW + 2 * XM)), jnp.bfloat16),
            pltpu.VMEM((9 * C, BN * (DHW + 2 * HW)), jnp.bfloat16),
        ],
        compiler_params=pltpu.CompilerParams(
            dimension_semantics=("parallel",)),
    )(x_in, w3, scale, shift)


def _finish_body(x_ref, res_ref, a_ref, b_ref, o_ref, *, D, HW):
    z = x_ref[0].astype(jnp.float32) * a_ref[...] + b_ref[...]
    for d in range(D):
        y = z[:, d * HW:(d + 1) * HW] + res_ref[0, :, d, :]
        o_ref[0, :, d, :] = jnp.maximum(y, 0.0)


def _finish(x_out2, res4, scale, shift, D, HW):
    N, C, DHW = x_out2.shape
    return pl.pallas_call(
        functools.partial(_finish_body, D=D, HW=HW),
        out_shape=jax.ShapeDtypeStruct((N, C, D, HW), jnp.float32),
        grid=(N,),
        in_specs=[
            pl.BlockSpec((1, C, DHW), lambda n: (n, 0, 0)),
            pl.BlockSpec((1, C, D, HW), lambda n: (n, 0, 0, 0)),
            pl.BlockSpec((C, 1), lambda n: (0, 0)),
            pl.BlockSpec((C, 1), lambda n: (0, 0)),
        ],
        out_specs=pl.BlockSpec((1, C, D, HW), lambda n: (n, 0, 0, 0)),
        compiler_params=pltpu.CompilerParams(
            dimension_semantics=("parallel",)),
    )(x_out2, res4, scale, shift)


def _fold_bn(sums, ssqs, gamma, beta, count):
    s = jnp.sum(sums, axis=0).reshape(-1)
    q = jnp.sum(ssqs, axis=0).reshape(-1)
    mean = s / count
    var = jnp.maximum(q / count - mean * mean, 0.0)
    scale = gamma * jax.lax.rsqrt(var + _EPS)
    shift = beta - mean * scale
    return scale.reshape(-1, 1), shift.reshape(-1, 1)


def _pack_w(w):
    Cout, Cin = w.shape[:2]
    return jnp.transpose(w, (2, 0, 3, 4, 1)).reshape(3 * Cout, 9 * Cin) \
        .astype(jnp.bfloat16)


@jax.jit
def kernel(x, w1, w2, g1, g2, b1, b2):
    N, C, D, H, W = x.shape
    DHW = D * H * W
    HW = H * W
    x4 = x.reshape(N, C, D, HW)

    w1p = _pack_w(w1)
    w2p = _pack_w(w2)
    ones = jnp.ones((C, 1), jnp.float32)
    zeros = jnp.zeros((C, 1), jnp.float32)

    out1, s1, q1 = _conv9(x4, w1p, D, H, W, ones, zeros,
                          apply_affine=False, planar_in=True)
    scale1, shift1 = _fold_bn(s1, q1, g1, b1, N * DHW)

    out2, s2, q2 = _conv9(out1, w2p, D, H, W, scale1, shift1,
                          apply_affine=True)
    scale2, shift2 = _fold_bn(s2, q2, g2, b2, N * DHW)

    y = _finish(out2, x4.astype(jnp.float32), scale2, shift2, D, HW)
    return y.reshape(N, C, D, H, W)


# speedup vs baseline: 1.0168x; 1.0168x over previous
import functools

import jax
import jax.numpy as jnp
from jax.experimental import pallas as pl
from jax.experimental.pallas import tpu as pltpu

_EPS = 1e-5


def _conv9_body(x_ref, w_ref, a_ref, b_ref,
                o_ref, st_ref,
                xp_ref, col_ref,
                *, D, H, W, XM, BN, apply_affine, planar_in):
    C = x_ref.shape[1]
    DHW = D * H * W
    HW = H * W
    SEG = DHW + 2 * XM
    CSEG = DHW + 2 * HW

    zx = jnp.zeros((C, XM), jnp.bfloat16)
    zc = jnp.zeros((9 * C, HW), jnp.bfloat16)
    for b in range(BN):
        xp_ref[:, b * SEG:b * SEG + XM] = zx
        xp_ref[:, b * SEG + XM + DHW:(b + 1) * SEG] = zx
        col_ref[:, b * CSEG:b * CSEG + HW] = zc
        col_ref[:, b * CSEG + HW + DHW:(b + 1) * CSEG] = zc

    for b in range(BN):
        base = b * SEG + XM
        if planar_in:
            for d in range(D):
                xp_ref[:, base + d * HW:base + (d + 1) * HW] = \
                    x_ref[b, :, d, :].astype(jnp.bfloat16)
        else:
            x = x_ref[b]
            if apply_affine:
                x = jnp.maximum(
                    x.astype(jnp.float32) * a_ref[...] + b_ref[...], 0.0)
            xp_ref[:, base:base + DHW] = x.astype(jnp.bfloat16)

    pos = jax.lax.broadcasted_iota(jnp.int32, (1, DHW), 1)
    h_pos = (pos // W) % H
    w_pos = pos % W
    hm = {-1: (h_pos >= 1).astype(jnp.bfloat16),
          0: None,
          1: (h_pos <= H - 2).astype(jnp.bfloat16)}
    wm = {-1: (w_pos >= 1).astype(jnp.bfloat16),
          0: None,
          1: (w_pos <= W - 2).astype(jnp.bfloat16)}

    for b in range(BN):
        base = b * SEG + XM
        t = 0
        for oh in (-1, 0, 1):
            for ow in (-1, 0, 1):
                si = oh * W + ow
                xs = xp_ref[:, base + si:base + si + DHW]
                m = hm[oh] if wm[ow] is None else (
                    wm[ow] if hm[oh] is None else hm[oh] * wm[ow])
                if m is not None:
                    xs = xs * m
                col_ref[t * C:(t + 1) * C,
                        b * CSEG + HW:b * CSEG + HW + DHW] = xs
                t += 1

    Cout = o_ref.shape[1]
    p = jnp.dot(w_ref[...], col_ref[...],
                preferred_element_type=jnp.float32)
    for b in range(BN):
        cb = b * CSEG
        acc = (p[0:Cout, cb:cb + DHW]
               + p[Cout:2 * Cout, cb + HW:cb + HW + DHW]
               + p[2 * Cout:3 * Cout, cb + 2 * HW:cb + 2 * HW + DHW])
        o_ref[b] = acc.astype(jnp.bfloat16)
        st_ref[b] = jnp.concatenate(
            [jnp.sum(acc, axis=1, keepdims=True),
             jnp.sum(acc * acc, axis=1, keepdims=True)], axis=1)


def _conv9(x_in, w3, D, H, W, scale, shift, apply_affine, planar_in=False,
           BN=1):
    N, C = x_in.shape[:2]
    DHW = D * H * W
    Cout = w3.shape[0] // 3
    HW = H * W
    XM = 128
    assert XM >= W + 1 and N % BN == 0

    if planar_in:
        x_spec = pl.BlockSpec((BN, C, D, HW), lambda n: (n, 0, 0, 0))
    else:
        x_spec = pl.BlockSpec((BN, C, DHW), lambda n: (n, 0, 0))
    body = functools.partial(_conv9_body, D=D, H=H, W=W, XM=XM, BN=BN,
                             apply_affine=apply_affine, planar_in=planar_in)
    return pl.pallas_call(
        body,
        out_shape=(
            jax.ShapeDtypeStruct((N, Cout, DHW), jnp.bfloat16),
            jax.ShapeDtypeStruct((N, Cout, 2), jnp.float32),
        ),
        grid=(N // BN,),
        in_specs=[
            x_spec,
            pl.BlockSpec((3 * Cout, 9 * C), lambda n: (0, 0)),
            pl.BlockSpec((C, 1), lambda n: (0, 0)),
            pl.BlockSpec((C, 1), lambda n: (0, 0)),
        ],
        out_specs=(
            pl.BlockSpec((BN, Cout, DHW), lambda n: (n, 0, 0)),
            pl.BlockSpec((BN, Cout, 2), lambda n: (n, 0, 0)),
        ),
        scratch_shapes=[
            pltpu.VMEM((C, BN * (DHW + 2 * XM)), jnp.bfloat16),
            pltpu.VMEM((9 * C, BN * (DHW + 2 * HW)), jnp.bfloat16),
        ],
        compiler_params=pltpu.CompilerParams(
            dimension_semantics=("parallel",)),
    )(x_in, w3, scale, shift)


def _finish_body(x_ref, res_ref, a_ref, b_ref, o_ref, *, D, HW):
    z = x_ref[0].astype(jnp.float32) * a_ref[...] + b_ref[...]
    for d in range(D):
        y = z[:, d * HW:(d + 1) * HW] + res_ref[0, :, d, :]
        o_ref[0, :, d, :] = jnp.maximum(y, 0.0)


def _finish(x_out2, res4, scale, shift, D, HW):
    N, C, DHW = x_out2.shape
    return pl.pallas_call(
        functools.partial(_finish_body, D=D, HW=HW),
        out_shape=jax.ShapeDtypeStruct((N, C, D, HW), jnp.float32),
        grid=(N,),
        in_specs=[
            pl.BlockSpec((1, C, DHW), lambda n: (n, 0, 0)),
            pl.BlockSpec((1, C, D, HW), lambda n: (n, 0, 0, 0)),
            pl.BlockSpec((C, 1), lambda n: (0, 0)),
            pl.BlockSpec((C, 1), lambda n: (0, 0)),
        ],
        out_specs=pl.BlockSpec((1, C, D, HW), lambda n: (n, 0, 0, 0)),
        compiler_params=pltpu.CompilerParams(
            dimension_semantics=("parallel",)),
    )(x_out2, res4, scale, shift)


def _fold_bn(stats, gamma, beta, count):
    s = jnp.sum(stats[:, :, 0], axis=0).reshape(-1)
    q = jnp.sum(stats[:, :, 1], axis=0).reshape(-1)
    mean = s / count
    var = jnp.maximum(q / count - mean * mean, 0.0)
    scale = gamma * jax.lax.rsqrt(var + _EPS)
    shift = beta - mean * scale
    return scale.reshape(-1, 1), shift.reshape(-1, 1)


def _pack_w(w):
    Cout, Cin = w.shape[:2]
    return jnp.transpose(w, (2, 0, 3, 4, 1)).reshape(3 * Cout, 9 * Cin) \
        .astype(jnp.bfloat16)


@jax.jit
def kernel(x, w1, w2, g1, g2, b1, b2):
    N, C, D, H, W = x.shape
    DHW = D * H * W
    HW = H * W
    x4 = x.reshape(N, C, D, HW)

    w1p = _pack_w(w1)
    w2p = _pack_w(w2)
    ones = jnp.ones((C, 1), jnp.float32)
    zeros = jnp.zeros((C, 1), jnp.float32)

    out1, st1 = _conv9(x4, w1p, D, H, W, ones, zeros,
                       apply_affine=False, planar_in=True)
    scale1, shift1 = _fold_bn(st1, g1, b1, N * DHW)

    out2, st2 = _conv9(out1, w2p, D, H, W, scale1, shift1,
                       apply_affine=True)
    scale2, shift2 = _fold_bn(st2, g2, b2, N * DHW)

    y = _finish(out2, x4.astype(jnp.float32), scale2, shift2, D, HW)
    return y.reshape(N, C, D, H, W)


# speedup vs baseline: 1.0585x; 1.0410x over previous
import functools

import jax
import jax.numpy as jnp
from jax.experimental import pallas as pl
from jax.experimental.pallas import tpu as pltpu

_EPS = 1e-5


def _conv9_body(x_ref, w_ref, a_ref, b_ref,
                o_ref, st_ref,
                xp_ref, col_ref,
                *, D, H, W, XM, BN, apply_affine, planar_in):
    C = x_ref.shape[1]
    DHW = D * H * W
    HW = H * W
    SEG = DHW + 2 * XM
    CSEG = DHW + 2 * HW

    zx = jnp.zeros((C, XM), jnp.bfloat16)
    zc = jnp.zeros((9 * C, HW), jnp.bfloat16)
    for b in range(BN):
        xp_ref[:, b * SEG:b * SEG + XM] = zx
        xp_ref[:, b * SEG + XM + DHW:(b + 1) * SEG] = zx
        col_ref[:, b * CSEG:b * CSEG + HW] = zc
        col_ref[:, b * CSEG + HW + DHW:(b + 1) * CSEG] = zc

    for b in range(BN):
        base = b * SEG + XM
        if planar_in:
            xp_ref[:, base:base + DHW] = \
                x_ref[b].reshape(C, DHW).astype(jnp.bfloat16)
        else:
            x = x_ref[b]
            if apply_affine:
                x = jnp.maximum(
                    x.astype(jnp.float32) * a_ref[...] + b_ref[...], 0.0)
            xp_ref[:, base:base + DHW] = x.astype(jnp.bfloat16)

    pos = jax.lax.broadcasted_iota(jnp.int32, (1, DHW), 1)
    h_pos = (pos // W) % H
    w_pos = pos % W
    hm = {-1: (h_pos >= 1).astype(jnp.bfloat16),
          0: None,
          1: (h_pos <= H - 2).astype(jnp.bfloat16)}
    wm = {-1: (w_pos >= 1).astype(jnp.bfloat16),
          0: None,
          1: (w_pos <= W - 2).astype(jnp.bfloat16)}

    for b in range(BN):
        base = b * SEG + XM
        t = 0
        for oh in (-1, 0, 1):
            for ow in (-1, 0, 1):
                si = oh * W + ow
                xs = xp_ref[:, base + si:base + si + DHW]
                m = hm[oh] if wm[ow] is None else (
                    wm[ow] if hm[oh] is None else hm[oh] * wm[ow])
                if m is not None:
                    xs = xs * m
                col_ref[t * C:(t + 1) * C,
                        b * CSEG + HW:b * CSEG + HW + DHW] = xs
                t += 1

    Cout = o_ref.shape[1]
    p = jnp.dot(w_ref[...], col_ref[...],
                preferred_element_type=jnp.float32)
    for b in range(BN):
        cb = b * CSEG
        acc = (p[0:Cout, cb:cb + DHW]
               + p[Cout:2 * Cout, cb + HW:cb + HW + DHW]
               + p[2 * Cout:3 * Cout, cb + 2 * HW:cb + 2 * HW + DHW])
        o_ref[b] = acc.astype(jnp.bfloat16)
        st_ref[b] = jnp.concatenate(
            [jnp.sum(acc, axis=1, keepdims=True),
             jnp.sum(acc * acc, axis=1, keepdims=True)], axis=1)


def _conv9(x_in, w3, D, H, W, scale, shift, apply_affine, planar_in=False,
           BN=1):
    N, C = x_in.shape[:2]
    DHW = D * H * W
    Cout = w3.shape[0] // 3
    HW = H * W
    XM = 128
    assert XM >= W + 1 and N % BN == 0

    if planar_in:
        x_spec = pl.BlockSpec((BN, C, D, HW), lambda n: (n, 0, 0, 0))
    else:
        x_spec = pl.BlockSpec((BN, C, DHW), lambda n: (n, 0, 0))
    body = functools.partial(_conv9_body, D=D, H=H, W=W, XM=XM, BN=BN,
                             apply_affine=apply_affine, planar_in=planar_in)
    return pl.pallas_call(
        body,
        out_shape=(
            jax.ShapeDtypeStruct((N, Cout, DHW), jnp.bfloat16),
            jax.ShapeDtypeStruct((N, Cout, 2), jnp.float32),
        ),
        grid=(N // BN,),
        in_specs=[
            x_spec,
            pl.BlockSpec((3 * Cout, 9 * C), lambda n: (0, 0)),
            pl.BlockSpec((C, 1), lambda n: (0, 0)),
            pl.BlockSpec((C, 1), lambda n: (0, 0)),
        ],
        out_specs=(
            pl.BlockSpec((BN, Cout, DHW), lambda n: (n, 0, 0)),
            pl.BlockSpec((BN, Cout, 2), lambda n: (n, 0, 0)),
        ),
        scratch_shapes=[
            pltpu.VMEM((C, BN * (DHW + 2 * XM)), jnp.bfloat16),
            pltpu.VMEM((9 * C, BN * (DHW + 2 * HW)), jnp.bfloat16),
        ],
        compiler_params=pltpu.CompilerParams(
            dimension_semantics=("parallel",)),
    )(x_in, w3, scale, shift)


def _finish_body(x_ref, res_ref, a_ref, b_ref, o_ref, *, D, HW):
    z = x_ref[0].astype(jnp.float32) * a_ref[...] + b_ref[...]
    for d in range(D):
        y = z[:, d * HW:(d + 1) * HW] + res_ref[0, :, d, :]
        o_ref[0, :, d, :] = jnp.maximum(y, 0.0)


def _finish(x_out2, res4, scale, shift, D, HW):
    N, C, DHW = x_out2.shape
    return pl.pallas_call(
        functools.partial(_finish_body, D=D, HW=HW),
        out_shape=jax.ShapeDtypeStruct((N, C, D, HW), jnp.float32),
        grid=(N,),
        in_specs=[
            pl.BlockSpec((1, C, DHW), lambda n: (n, 0, 0)),
            pl.BlockSpec((1, C, D, HW), lambda n: (n, 0, 0, 0)),
            pl.BlockSpec((C, 1), lambda n: (0, 0)),
            pl.BlockSpec((C, 1), lambda n: (0, 0)),
        ],
        out_specs=pl.BlockSpec((1, C, D, HW), lambda n: (n, 0, 0, 0)),
        compiler_params=pltpu.CompilerParams(
            dimension_semantics=("parallel",)),
    )(x_out2, res4, scale, shift)


def _fold_bn(stats, gamma, beta, count):
    s = jnp.sum(stats[:, :, 0], axis=0).reshape(-1)
    q = jnp.sum(stats[:, :, 1], axis=0).reshape(-1)
    mean = s / count
    var = jnp.maximum(q / count - mean * mean, 0.0)
    scale = gamma * jax.lax.rsqrt(var + _EPS)
    shift = beta - mean * scale
    return scale.reshape(-1, 1), shift.reshape(-1, 1)


def _pack_w(w):
    Cout, Cin = w.shape[:2]
    return jnp.transpose(w, (2, 0, 3, 4, 1)).reshape(3 * Cout, 9 * Cin) \
        .astype(jnp.bfloat16)


@jax.jit
def kernel(x, w1, w2, g1, g2, b1, b2):
    N, C, D, H, W = x.shape
    DHW = D * H * W
    HW = H * W
    x4 = x.reshape(N, C, D, HW)

    w1p = _pack_w(w1)
    w2p = _pack_w(w2)
    ones = jnp.ones((C, 1), jnp.float32)
    zeros = jnp.zeros((C, 1), jnp.float32)

    out1, st1 = _conv9(x4, w1p, D, H, W, ones, zeros,
                       apply_affine=False, planar_in=True)
    scale1, shift1 = _fold_bn(st1, g1, b1, N * DHW)

    out2, st2 = _conv9(out1, w2p, D, H, W, scale1, shift1,
                       apply_affine=True)
    scale2, shift2 = _fold_bn(st2, g2, b2, N * DHW)

    y = _finish(out2, x4.astype(jnp.float32), scale2, shift2, D, HW)
    return y.reshape(N, C, D, H, W)


# speedup vs baseline: 1.0770x; 1.0174x over previous
import functools

import jax
import jax.numpy as jnp
from jax.experimental import pallas as pl
from jax.experimental.pallas import tpu as pltpu

_EPS = 1e-5


def _conv9_body(x_ref, w_ref, a_ref, b_ref,
                o_ref, st_ref,
                xp_ref, col_ref,
                *, D, H, W, XM, BN, apply_affine, planar_in):
    C = x_ref.shape[1]
    DHW = D * H * W
    HW = H * W
    SEG = DHW + 2 * XM
    CSEG = DHW + 2 * HW

    zx = jnp.zeros((C, XM), jnp.bfloat16)
    zc = jnp.zeros((9 * C, HW), jnp.bfloat16)
    for b in range(BN):
        xp_ref[:, b * SEG:b * SEG + XM] = zx
        xp_ref[:, b * SEG + XM + DHW:(b + 1) * SEG] = zx
        col_ref[:, b * CSEG:b * CSEG + HW] = zc
        col_ref[:, b * CSEG + HW + DHW:(b + 1) * CSEG] = zc

    for b in range(BN):
        base = b * SEG + XM
        if planar_in:
            xp_ref[:, base:base + DHW] = \
                x_ref[b].reshape(C, DHW).astype(jnp.bfloat16)
        else:
            x = x_ref[b]
            if apply_affine:
                x = jnp.maximum(
                    x.astype(jnp.float32) * a_ref[...] + b_ref[...], 0.0)
            xp_ref[:, base:base + DHW] = x.astype(jnp.bfloat16)

    pos = jax.lax.broadcasted_iota(jnp.int32, (1, DHW), 1)
    h_pos = (pos // W) % H
    w_pos = pos % W
    hm = {-1: (h_pos >= 1).astype(jnp.bfloat16),
          0: None,
          1: (h_pos <= H - 2).astype(jnp.bfloat16)}
    wm = {-1: (w_pos >= 1).astype(jnp.bfloat16),
          0: None,
          1: (w_pos <= W - 2).astype(jnp.bfloat16)}

    for b in range(BN):
        base = b * SEG + XM
        t = 0
        for oh in (-1, 0, 1):
            for ow in (-1, 0, 1):
                si = oh * W + ow
                xs = xp_ref[:, base + si:base + si + DHW]
                m = hm[oh] if wm[ow] is None else (
                    wm[ow] if hm[oh] is None else hm[oh] * wm[ow])
                if m is not None:
                    xs = xs * m
                col_ref[t * C:(t + 1) * C,
                        b * CSEG + HW:b * CSEG + HW + DHW] = xs
                t += 1

    Cout = o_ref.shape[1]
    p = jnp.dot(w_ref[...], col_ref[...],
                preferred_element_type=jnp.float32)
    for b in range(BN):
        cb = b * CSEG
        acc = (p[0:Cout, cb:cb + DHW]
               + p[Cout:2 * Cout, cb + HW:cb + HW + DHW]
               + p[2 * Cout:3 * Cout, cb + 2 * HW:cb + 2 * HW + DHW])
        o_ref[b] = acc.astype(jnp.bfloat16)
        st_ref[b] = jnp.concatenate(
            [jnp.sum(acc, axis=1, keepdims=True),
             jnp.sum(acc * acc, axis=1, keepdims=True)], axis=1)


def _conv9(x_in, w3, D, H, W, scale, shift, apply_affine, planar_in=False,
           BN=1):
    N, C = x_in.shape[:2]
    DHW = D * H * W
    Cout = w3.shape[0] // 3
    HW = H * W
    XM = 128
    assert XM >= W + 1 and N % BN == 0

    if planar_in:
        x_spec = pl.BlockSpec((BN, C, D, HW), lambda n: (n, 0, 0, 0))
    else:
        x_spec = pl.BlockSpec((BN, C, DHW), lambda n: (n, 0, 0))
    body = functools.partial(_conv9_body, D=D, H=H, W=W, XM=XM, BN=BN,
                             apply_affine=apply_affine, planar_in=planar_in)
    return pl.pallas_call(
        body,
        out_shape=(
            jax.ShapeDtypeStruct((N, Cout, DHW), jnp.bfloat16),
            jax.ShapeDtypeStruct((N, Cout, 2), jnp.float32),
        ),
        grid=(N // BN,),
        in_specs=[
            x_spec,
            pl.BlockSpec((3 * Cout, 9 * C), lambda n: (0, 0)),
            pl.BlockSpec((C, 1), lambda n: (0, 0)),
            pl.BlockSpec((C, 1), lambda n: (0, 0)),
        ],
        out_specs=(
            pl.BlockSpec((BN, Cout, DHW), lambda n: (n, 0, 0)),
            pl.BlockSpec((BN, Cout, 2), lambda n: (n, 0, 0)),
        ),
        scratch_shapes=[
            pltpu.VMEM((C, BN * (DHW + 2 * XM)), jnp.bfloat16),
            pltpu.VMEM((9 * C, BN * (DHW + 2 * HW)), jnp.bfloat16),
        ],
        compiler_params=pltpu.CompilerParams(
            dimension_semantics=("parallel",)),
    )(x_in, w3, scale, shift)


def _finish_body(x_ref, res_ref, a_ref, b_ref, o_ref, *, D, HW):
    C, DHW = x_ref.shape[1], x_ref.shape[2]
    z = x_ref[0].astype(jnp.float32) * a_ref[...] + b_ref[...]
    y = jnp.maximum(z + res_ref[0].reshape(C, DHW), 0.0)
    o_ref[0] = y.reshape(C, D, HW)


def _finish(x_out2, res4, scale, shift, D, HW):
    N, C, DHW = x_out2.shape
    return pl.pallas_call(
        functools.partial(_finish_body, D=D, HW=HW),
        out_shape=jax.ShapeDtypeStruct((N, C, D, HW), jnp.float32),
        grid=(N,),
        in_specs=[
            pl.BlockSpec((1, C, DHW), lambda n: (n, 0, 0)),
            pl.BlockSpec((1, C, D, HW), lambda n: (n, 0, 0, 0)),
            pl.BlockSpec((C, 1), lambda n: (0, 0)),
            pl.BlockSpec((C, 1), lambda n: (0, 0)),
        ],
        out_specs=pl.BlockSpec((1, C, D, HW), lambda n: (n, 0, 0, 0)),
        compiler_params=pltpu.CompilerParams(
            dimension_semantics=("parallel",)),
    )(x_out2, res4, scale, shift)


def _fold_bn(stats, gamma, beta, count):
    s = jnp.sum(stats[:, :, 0], axis=0).reshape(-1)
    q = jnp.sum(stats[:, :, 1], axis=0).reshape(-1)
    mean = s / count
    var = jnp.maximum(q / count - mean * mean, 0.0)
    scale = gamma * jax.lax.rsqrt(var + _EPS)
    shift = beta - mean * scale
    return scale.reshape(-1, 1), shift.reshape(-1, 1)


def _pack_w(w):
    Cout, Cin = w.shape[:2]
    return jnp.transpose(w, (2, 0, 3, 4, 1)).reshape(3 * Cout, 9 * Cin) \
        .astype(jnp.bfloat16)


@jax.jit
def kernel(x, w1, w2, g1, g2, b1, b2):
    N, C, D, H, W = x.shape
    DHW = D * H * W
    HW = H * W
    x4 = x.reshape(N, C, D, HW)

    w1p = _pack_w(w1)
    w2p = _pack_w(w2)
    ones = jnp.ones((C, 1), jnp.float32)
    zeros = jnp.zeros((C, 1), jnp.float32)

    out1, st1 = _conv9(x4, w1p, D, H, W, ones, zeros,
                       apply_affine=False, planar_in=True)
    scale1, shift1 = _fold_bn(st1, g1, b1, N * DHW)

    out2, st2 = _conv9(out1, w2p, D, H, W, scale1, shift1,
                       apply_affine=True)
    scale2, shift2 = _fold_bn(st2, g2, b2, N * DHW)

    y = _finish(out2, x4.astype(jnp.float32), scale2, shift2, D, HW)
    return y.reshape(N, C, D, H, W)
